# baseline (speedup 1.0000x reference)
import jax
import jax.numpy as jnp
from jax import lax
from jax.experimental import pallas as pl
from jax.experimental.pallas import tpu as pltpu
from jax.experimental.pallas import tpu_sc as plsc

_B, _H, _W, _C = 4, 384, 384, 96
_N = _B * _H * _W
_NC, _NS, _L = 2, 16, 16
_NW = _NC * _NS
_ROWS_W = _H * _B // _NW
_CHUNK = 64
_CPR = _W // _CHUNK
_G = _CHUNK // _L


def _warp_body(img_hbm, fx_hbm, fy_hbm, out_hbm, fx_v, fy_v, wx_v, wy_v,
               i00, i01, i10, i11, r00, r01, r10, r11, out_v, sem):
    wid = lax.axis_index("s") * _NC + lax.axis_index("c")
    b = wid // 8
    h0 = wid * _ROWS_W - b * _H
    boff = b * (_H * _W)
    pix_g = [lax.iota(jnp.int32, _L) + g * _L for g in range(_G)]

    def row_body(r, carry):
        h = h0 + r
        rowbase = boff + h * _W

        def chunk_body(u, carry2):
            base = rowbase + u * _CHUNK
            pltpu.sync_copy(fx_hbm.at[pl.ds(base, _CHUNK)], fx_v)
            pltpu.sync_copy(fy_hbm.at[pl.ds(base, _CHUNK)], fy_v)
            hf = jnp.full((_L,), h, jnp.int32).astype(jnp.float32)
            for g in range(_G):
                sl = pl.ds(g * _L, _L)
                wcol = pix_g[g] + u * _CHUNK
                qx = jnp.clip(wcol.astype(jnp.float32) + fx_v[sl],
                              0.0, float(_W - 1))
                qy = jnp.clip(hf + fy_v[sl], 0.0, float(_H - 1))
                x0 = jnp.minimum(qx.astype(jnp.int32), _W - 2)
                y0 = jnp.minimum(qy.astype(jnp.int32), _H - 2)
                ax = jnp.minimum(qx - x0.astype(jnp.float32), 1.0)
                ay = jnp.minimum(qy - y0.astype(jnp.float32), 1.0)
                l00 = boff + y0 * _W + x0
                i00[sl] = l00
                i01[sl] = l00 + 1
                i10[sl] = l00 + _W
                i11[sl] = l00 + _W + 1
                wx_v[sl] = ax
                wy_v[sl] = ay
            copies = [pltpu.async_copy(img_hbm.at[iv], rv, sem)
                      for iv, rv in ((i00, r00), (i01, r01),
                                     (i10, r10), (i11, r11))]
            for cp in copies:
                cp.wait()
            def pbody(p, p_carry):
                pp = jnp.full((_L,), p, jnp.int32)
                wxp = plsc.load_gather(wx_v, [pp])
                wyp = plsc.load_gather(wy_v, [pp])
                for j in range(_C // _L):
                    cs = pl.ds(j * _L, _L)
                    v00 = r00[p, cs]
                    v01 = r01[p, cs]
                    v10 = r10[p, cs]
                    v11 = r11[p, cs]
                    top = v00 + wxp * (v01 - v00)
                    bot = v10 + wxp * (v11 - v10)
                    out_v[p, cs] = top + wyp * (bot - top)
                return p_carry

            lax.fori_loop(0, _CHUNK, pbody, 0, unroll=2)
            pltpu.sync_copy(out_v, out_hbm.at[pl.ds(base, _CHUNK)])
            return carry2

        lax.fori_loop(0, _CPR, chunk_body, 0)
        return carry

    lax.fori_loop(0, _ROWS_W, row_body, 0)


@jax.jit
def kernel(img, flo):
    imgf = img.reshape(_N, _C)
    flof = flo.reshape(_N, 2)
    fx = flof[:, 0]
    fy = flof[:, 1]
    mesh = plsc.VectorSubcoreMesh(core_axis_name="c", subcore_axis_name="s")
    out = pl.kernel(
        _warp_body,
        out_type=jax.ShapeDtypeStruct((_N, _C), jnp.float32),
        mesh=mesh,
        compiler_params=pltpu.CompilerParams(use_tc_tiling_on_sc=False,
                                             needs_layout_passes=False),
        scratch_types=[
            pltpu.VMEM((_CHUNK,), jnp.float32),
            pltpu.VMEM((_CHUNK,), jnp.float32),
            pltpu.VMEM((_CHUNK,), jnp.float32),
            pltpu.VMEM((_CHUNK,), jnp.float32),
            pltpu.VMEM((_CHUNK,), jnp.int32),
            pltpu.VMEM((_CHUNK,), jnp.int32),
            pltpu.VMEM((_CHUNK,), jnp.int32),
            pltpu.VMEM((_CHUNK,), jnp.int32),
            pltpu.VMEM((_CHUNK, _C), jnp.float32),
            pltpu.VMEM((_CHUNK, _C), jnp.float32),
            pltpu.VMEM((_CHUNK, _C), jnp.float32),
            pltpu.VMEM((_CHUNK, _C), jnp.float32),
            pltpu.VMEM((_CHUNK, _C), jnp.float32),
            pltpu.SemaphoreType.DMA,
        ],
    )(imgf, fx, fy)
    return out.reshape(_B, _H, _W, _C)
